# baseline (speedup 1.0000x reference)
import functools

import jax
import jax.numpy as jnp
from jax import lax
from jax.experimental import pallas as pl
from jax.experimental.pallas import tpu as pltpu
from jax.experimental.pallas import tpu_sc as plsc


def _rank_body(nrow_ref, restore_ref, mask_ref, keep_ref, *, L, K, R):
    i = pl.program_id(0)
    rows = nrow_ref[...].reshape(R, L)
    colsT = jnp.transpose(rows)
    ki = lax.broadcasted_iota(jnp.int32, (L, L), 0)
    ji = lax.broadcasted_iota(jnp.int32, (L, L), 1)
    kltj = ki < ji
    ranks = []
    for r in range(R):
        row = rows[r:r + 1, :]
        col = colsT[:, r:r + 1]
        lt = col < row
        eq = col == row
        prec = jnp.logical_or(lt, jnp.logical_and(eq, kltj))
        ranks.append(jnp.sum(prec.astype(jnp.int32), axis=0, keepdims=True))
    rank = jnp.concatenate(ranks, axis=0)
    restore_ref[...] = rank.reshape(1, R, L)
    mask_ref[...] = (rank >= K).astype(jnp.float32).reshape(1, R, L)
    rank3 = rank.reshape(R, 1, L)
    ri = lax.broadcasted_iota(jnp.int32, (R, K, L), 1)
    hit = rank3 == ri
    joff = lax.broadcasted_iota(jnp.int32, (R, K, L), 2) + (
        lax.broadcasted_iota(jnp.int32, (R, K, L), 0) + i * R) * L
    keep = jnp.sum(jnp.where(hit, joff, 0), axis=2)
    keep_ref[...] = keep.reshape(1, R, K)


def _make_rank_call(N, L, K, R):
    body = functools.partial(_rank_body, L=L, K=K, R=R)
    G = N // R
    return pl.pallas_call(
        body,
        grid=(G,),
        in_specs=[
            pl.BlockSpec((1, R, L), lambda i: (i, 0, 0)),
        ],
        out_specs=[
            pl.BlockSpec((1, R, L), lambda i: (i, 0, 0)),
            pl.BlockSpec((1, R, L), lambda i: (i, 0, 0)),
            pl.BlockSpec((1, R, K), lambda i: (i, 0, 0)),
        ],
        out_shape=[
            jax.ShapeDtypeStruct((G, R, L), jnp.int32),
            jax.ShapeDtypeStruct((G, R, L), jnp.float32),
            jax.ShapeDtypeStruct((G, R, K), jnp.int32),
        ],
    )


def _make_gather_call(N, L, D, K, R):
    info = plsc.get_sparse_core_info()
    NC, NS = info.num_cores, info.num_subcores
    NW = NC * NS
    G = N // R
    ROWS_W = N // NW
    b_per_w = ROWS_W * K
    CH = 32
    NBUF = 4
    assert b_per_w % CH == 0
    NCH = b_per_w // CH
    CPR = K // CH
    assert NCH >= NBUF
    mesh = plsc.VectorSubcoreMesh(core_axis_name="c", subcore_axis_name="s")

    @functools.partial(
        pl.kernel,
        mesh=mesh,
        out_type=jax.ShapeDtypeStruct((N * K, D), jnp.float32),
        scratch_types=[
            pltpu.VMEM((ROWS_W, K), jnp.int32),
            [pltpu.VMEM((CH, D), jnp.float32)] * NBUF,
            [pltpu.SemaphoreType.DMA] * NBUF,
            [pltpu.SemaphoreType.DMA] * NBUF,
        ],
    )
    def gather_k(x_hbm, idx_hbm, out_hbm, idx_v, bufs, gsem, wsem):
        wid = lax.axis_index("s") * NC + lax.axis_index("c")
        base = wid * b_per_w
        wpb = R // ROWS_W
        gw = lax.shift_right_logical(wid, wpb.bit_length() - 1)
        rw = jnp.bitwise_and(wid, wpb - 1) * ROWS_W
        pltpu.sync_copy(idx_hbm.at[gw, pl.ds(rw, ROWS_W)], idx_v)

        def idx_slice(c):
            return idx_v.at[c // CPR, pl.ds((c % CPR) * CH, CH)]

        gath = [None] * NBUF
        wrt = [None] * NBUF
        for c in range(2):
            gath[c] = pltpu.async_copy(
                x_hbm.at[idx_slice(c)], bufs[c], gsem[c])
        for c in range(NCH):
            b = c % NBUF
            gath[b].wait()
            wrt[b] = pltpu.async_copy(
                bufs[b], out_hbm.at[pl.ds(base + c * CH, CH)], wsem[b])
            g = c + 2
            if g < NCH:
                bg = g % NBUF
                if wrt[bg] is not None:
                    wrt[bg].wait()
                gath[bg] = pltpu.async_copy(
                    x_hbm.at[idx_slice(g)], bufs[bg], gsem[bg])
        wrt[(NCH - 2) % NBUF].wait()
        wrt[(NCH - 1) % NBUF].wait()

    return gather_k


def kernel(x, noise):
    N, L, D = x.shape
    K = L - int(L * 0.75)
    R = 16
    rank_call = _make_rank_call(N, L, K, R)
    restore3, mask3, keep3 = rank_call(noise.reshape(N // R, R, L))
    ids_restore = restore3.reshape(N, L)
    mask = mask3.reshape(N, L)

    gather_k = _make_gather_call(N, L, D, K, R)
    x_masked = gather_k(x.reshape(N * L, D), keep3)
    return x_masked.reshape(N, K, D), mask, ids_restore


# speedup vs baseline: 1.0175x; 1.0175x over previous
import functools

import jax
import jax.numpy as jnp
from jax import lax
from jax.experimental import pallas as pl
from jax.experimental.pallas import tpu as pltpu
from jax.experimental.pallas import tpu_sc as plsc


def _rank_body(nrow_ref, restore_ref, mask_ref, keep_ref, *, L, K, R):
    i = pl.program_id(0)
    rows = nrow_ref[...].reshape(R, L)
    colsT = jnp.transpose(rows)
    ki = lax.broadcasted_iota(jnp.int32, (L, L), 0)
    ji = lax.broadcasted_iota(jnp.int32, (L, L), 1)
    kltj = ki < ji
    ranks = []
    for r in range(R):
        row = rows[r:r + 1, :]
        col = colsT[:, r:r + 1]
        lt = col < row
        eq = col == row
        prec = jnp.logical_or(lt, jnp.logical_and(eq, kltj))
        ranks.append(jnp.sum(prec.astype(jnp.int32), axis=0, keepdims=True))
    rank = jnp.concatenate(ranks, axis=0)
    restore_ref[...] = rank.reshape(1, R, L)
    mask_ref[...] = (rank >= K).astype(jnp.float32).reshape(1, R, L)
    rank3 = rank.reshape(R, 1, L)
    ri = lax.broadcasted_iota(jnp.int32, (R, K, L), 1)
    hit = rank3 == ri
    joff = lax.broadcasted_iota(jnp.int32, (R, K, L), 2) + (
        lax.broadcasted_iota(jnp.int32, (R, K, L), 0) + i * R) * L
    keep = jnp.sum(jnp.where(hit, joff, 0), axis=2)
    keep_ref[...] = keep.reshape(1, R, K)


def _make_rank_call(N, L, K, R):
    body = functools.partial(_rank_body, L=L, K=K, R=R)
    G = N // R
    return pl.pallas_call(
        body,
        grid=(G,),
        in_specs=[
            pl.BlockSpec((1, R, L), lambda i: (i, 0, 0)),
        ],
        out_specs=[
            pl.BlockSpec((1, R, L), lambda i: (i, 0, 0)),
            pl.BlockSpec((1, R, L), lambda i: (i, 0, 0)),
            pl.BlockSpec((1, R, K), lambda i: (i, 0, 0)),
        ],
        out_shape=[
            jax.ShapeDtypeStruct((G, R, L), jnp.int32),
            jax.ShapeDtypeStruct((G, R, L), jnp.float32),
            jax.ShapeDtypeStruct((G, R, K), jnp.int32),
        ],
    )


def _make_gather_call(N, L, D, K, R):
    info = plsc.get_sparse_core_info()
    NC, NS = info.num_cores, info.num_subcores
    NW = NC * NS
    G = N // R
    ROWS_W = N // NW
    b_per_w = ROWS_W * K
    CH = 32
    NBUF = 5
    AHEAD = 3
    assert b_per_w % CH == 0
    NCH = b_per_w // CH
    CPR = K // CH
    assert NCH >= NBUF
    mesh = plsc.VectorSubcoreMesh(core_axis_name="c", subcore_axis_name="s")

    @functools.partial(
        pl.kernel,
        mesh=mesh,
        out_type=jax.ShapeDtypeStruct((N * K, D), jnp.float32),
        scratch_types=[
            pltpu.VMEM((ROWS_W, K), jnp.int32),
            [pltpu.VMEM((CH, D), jnp.float32)] * NBUF,
            [pltpu.SemaphoreType.DMA] * NBUF,
            [pltpu.SemaphoreType.DMA] * NBUF,
        ],
    )
    def gather_k(x_hbm, idx_hbm, out_hbm, idx_v, bufs, gsem, wsem):
        wid = lax.axis_index("s") * NC + lax.axis_index("c")
        base = wid * b_per_w
        wpb = R // ROWS_W
        gw = lax.shift_right_logical(wid, wpb.bit_length() - 1)
        rw = jnp.bitwise_and(wid, wpb - 1) * ROWS_W
        pltpu.sync_copy(idx_hbm.at[gw, pl.ds(rw, ROWS_W)], idx_v)

        def idx_slice(c):
            return idx_v.at[c // CPR, pl.ds((c % CPR) * CH, CH)]

        gath = [None] * NBUF
        wrt = [None] * NBUF
        for c in range(AHEAD):
            gath[c] = pltpu.async_copy(
                x_hbm.at[idx_slice(c)], bufs[c], gsem[c])
        for c in range(NCH):
            b = c % NBUF
            gath[b].wait()
            wrt[b] = pltpu.async_copy(
                bufs[b], out_hbm.at[pl.ds(base + c * CH, CH)], wsem[b])
            g = c + AHEAD
            if g < NCH:
                bg = g % NBUF
                if wrt[bg] is not None:
                    wrt[bg].wait()
                gath[bg] = pltpu.async_copy(
                    x_hbm.at[idx_slice(g)], bufs[bg], gsem[bg])
        for k in range(max(0, NCH - NBUF), NCH):
            wrt[k % NBUF].wait()

    return gather_k


def kernel(x, noise):
    N, L, D = x.shape
    K = L - int(L * 0.75)
    R = 16
    rank_call = _make_rank_call(N, L, K, R)
    restore3, mask3, keep3 = rank_call(noise.reshape(N // R, R, L))
    ids_restore = restore3.reshape(N, L)
    mask = mask3.reshape(N, L)

    gather_k = _make_gather_call(N, L, D, K, R)
    x_masked = gather_k(x.reshape(N * L, D), keep3)
    return x_masked.reshape(N, K, D), mask, ids_restore
